# baseline (speedup 1.0000x reference)
import functools

import jax
import jax.numpy as jnp
from jax import lax
from jax.experimental import pallas as pl
from jax.experimental.pallas import tpu as pltpu
from jax.experimental.pallas import tpu_sc as plsc

_N = 10000
_E = 320000
_D_IN = 128
_HID = 256
_HEADS = 8
_T_OUT = 5
_NPD = 10240
_EP = 344064
_NC = 2
_NS = 16
_CA = 256
_CB1 = 64
_CB2 = 16

_sc_params = pltpu.CompilerParams(use_tc_tiling_on_sc=False,
                                  needs_layout_passes=False)


@functools.lru_cache(maxsize=1)
def _get_mesh():
    return plsc.VectorSubcoreMesh(core_axis_name="c", subcore_axis_name="s",
                                  num_cores=_NC, num_subcores=_NS)


def _f32(shape):
    return jax.ShapeDtypeStruct(shape, jnp.float32)


def _mm_body(a_ref, b_ref, bias_ref, o_ref):
    o_ref[...] = (jnp.dot(a_ref[...], b_ref[...], preferred_element_type=jnp.float32)
                  + bias_ref[...])


def _mm(a, b, bias, blk=1024):
    m, k = a.shape
    _, n = b.shape
    return pl.pallas_call(
        _mm_body,
        grid=(m // blk,),
        in_specs=[
            pl.BlockSpec((blk, k), lambda i: (i, 0)),
            pl.BlockSpec((k, n), lambda i: (0, 0)),
            pl.BlockSpec((1, n), lambda i: (0, 0)),
        ],
        out_specs=pl.BlockSpec((blk, n), lambda i: (i, 0)),
        out_shape=_f32((m, n)),
    )(a, b, bias.reshape(1, n))


def _rden_body(d_ref, o_ref):
    s = d_ref[0] + d_ref[1]
    o_ref[...] = 1.0 / jnp.maximum(s, 1e-30)


def _rden(denp):
    d = denp.reshape(2, _NPD * 16 // 128, 128)
    r = pl.pallas_call(
        _rden_body,
        grid=(4,),
        in_specs=[pl.BlockSpec((2, _NPD * 4 // 128, 128), lambda i: (0, i, 0))],
        out_specs=pl.BlockSpec((_NPD * 4 // 128, 128), lambda i: (i, 0)),
        out_shape=_f32((_NPD * 16 // 128, 128)),
    )(d)
    return r.reshape(_NPD, 16)


def _head_loss_body(h2_ref, b2_ref, wt1_ref, bt1_ref, wt2_ref, bt2_ref,
                    wc1_ref, bc1_ref, wc2_ref, bc2_ref, meta_ref, o_ref):
    i = pl.program_id(0)
    h2 = h2_ref[...] * (1.0 / _HEADS) + b2_ref[...]
    t1 = jnp.maximum(jnp.dot(h2, wt1_ref[...], preferred_element_type=jnp.float32)
                     + bt1_ref[...], 0.0)
    tl = jnp.dot(t1, wt2_ref[...], preferred_element_type=jnp.float32) + bt2_ref[...]
    c1 = jnp.maximum(jnp.dot(h2, wc1_ref[...], preferred_element_type=jnp.float32)
                     + bc1_ref[...], 0.0)
    cl = jnp.dot(c1, wc2_ref[...], preferred_element_type=jnp.float32) + bc2_ref[...]
    meta = meta_ref[...]
    tt = meta[:, 0:1]
    nt = meta[:, 1:2]
    msk = meta[:, 2:3]
    valid = meta[:, 3:4]
    colid = lax.broadcasted_iota(jnp.int32, tl.shape, 1).astype(jnp.float32)
    neg = jnp.float32(-1e30)

    tl5 = jnp.where(colid < 5.0, tl, neg)
    tmax = jnp.max(tl5, axis=1, keepdims=True)
    tsum = jnp.sum(jnp.where(colid < 5.0, jnp.exp(tl5 - tmax), 0.0), axis=1, keepdims=True)
    tlse = tmax + jnp.log(tsum)
    tsel = jnp.sum(jnp.where(colid == tt, tl, 0.0), axis=1, keepdims=True)
    ts_ce = (tlse - tsel) * valid

    cl2 = jnp.where(colid < 2.0, cl, neg)
    cmax = jnp.max(cl2, axis=1, keepdims=True)
    csum = jnp.sum(jnp.where(colid < 2.0, jnp.exp(cl2 - cmax), 0.0), axis=1, keepdims=True)
    clse = cmax + jnp.log(csum)
    csel = jnp.sum(jnp.where(colid == nt, cl, 0.0), axis=1, keepdims=True)
    cls_ce = (clse - csel) * msk * valid

    ts_s = jnp.sum(ts_ce)
    cls_s = jnp.sum(cls_ce)
    m_s = jnp.sum(msk * valid)
    ci = lax.broadcasted_iota(jnp.int32, (1, 128), 1).astype(jnp.float32)
    row = (jnp.where(ci == 0.0, ts_s, 0.0)
           + jnp.where(ci == 1.0, cls_s, 0.0)
           + jnp.where(ci == 2.0, m_s, 0.0))

    @pl.when(i == 0)
    def _():
        o_ref[...] = row

    @pl.when(i > 0)
    def _():
        o_ref[...] = o_ref[...] + row


def _head_loss(h2p, b2, wt1, bt1, wt2p, bt2p, wc1, bc1, wc2p, bc2p, meta, blk=1024):
    vec = lambda v: v.reshape(1, -1)
    return pl.pallas_call(
        _head_loss_body,
        grid=(_NPD // blk,),
        in_specs=[
            pl.BlockSpec((blk, _HID), lambda i: (i, 0)),
            pl.BlockSpec((1, _HID), lambda i: (0, 0)),
            pl.BlockSpec((_HID, _HID), lambda i: (0, 0)),
            pl.BlockSpec((1, _HID), lambda i: (0, 0)),
            pl.BlockSpec((_HID, 128), lambda i: (0, 0)),
            pl.BlockSpec((1, 128), lambda i: (0, 0)),
            pl.BlockSpec((_HID, _HID), lambda i: (0, 0)),
            pl.BlockSpec((1, _HID), lambda i: (0, 0)),
            pl.BlockSpec((_HID, 128), lambda i: (0, 0)),
            pl.BlockSpec((1, 128), lambda i: (0, 0)),
            pl.BlockSpec((blk, 128), lambda i: (i, 0)),
        ],
        out_specs=pl.BlockSpec((1, 128), lambda i: (0, 0)),
        out_shape=_f32((1, 128)),
    )(h2p, vec(b2), wt1, vec(bt1), wt2p, vec(bt2p), wc1, vec(bc1), wc2p, vec(bc2p), meta)


def _zero_vmem(ref, rows, width):
    z = jnp.zeros((16,), jnp.float32)
    def body(i):
        r = i // (width // 16)
        v = i % (width // 16)
        ref[r, pl.ds(v * 16, 16)] = z
    pl.loop(0, rows * (width // 16))(body)


@functools.lru_cache(maxsize=1)
def _make_sc_edge_w():
    return functools.partial(
        pl.kernel,
        out_type=(_f32((_EP, 16)), _f32((_NC, _NPD, 16))),
        mesh=_get_mesh(),
        compiler_params=_sc_params,
        scratch_types=dict(
            sidx=pltpu.VMEM((2, _CA), jnp.int32),
            didx=pltpu.VMEM((2, _CA), jnp.int32),
            bufS=pltpu.VMEM((2, _CA, 16), jnp.float32),
            bufD=pltpu.VMEM((2, _CA, 16), jnp.float32),
            wbuf=pltpu.VMEM((_CA, 16), jnp.float32),
            stage=pltpu.VMEM((_NPD // _NS, 16), jnp.float32),
            den_sh=pltpu.VMEM_SHARED((_NPD, 16), jnp.float32),
            semI0=pltpu.SemaphoreType.DMA,
            semI1=pltpu.SemaphoreType.DMA,
            semS0=pltpu.SemaphoreType.DMA,
            semS1=pltpu.SemaphoreType.DMA,
            semD0=pltpu.SemaphoreType.DMA,
            semD1=pltpu.SemaphoreType.DMA,
        ),
    )(_sc_edge_w_body)


def _sc_edge_w_body(ltabS, ltabD, srcp, dstp, w_out, den_out,
                    sidx, didx, bufS, bufD, wbuf, stage, den_sh,
                    semI0, semI1, semS0, semS1, semD0, semD1):
    c = lax.axis_index("c")
    s = lax.axis_index("s")
    wid = s * _NC + c
    per_worker = _EP // (_NC * _NS)
    nchunk = per_worker // _CA
    npairs = nchunk // 2
    rows_per_tile = _NPD // _NS
    semI = (semI0, semI1)
    semS = (semS0, semS1)
    semD = (semD0, semD1)

    _zero_vmem(stage, rows_per_tile, 16)
    pltpu.sync_copy(stage, den_sh.at[pl.ds(s * rows_per_tile, rows_per_tile)])
    plsc.subcore_barrier()

    lane = lax.iota(jnp.int32, 16)

    def issue_stage1(j, b):
        base = wid * per_worker + j * _CA
        pltpu.async_copy(srcp.at[pl.ds(base, _CA)], sidx.at[b], semI[b])
        pltpu.async_copy(dstp.at[pl.ds(base, _CA)], didx.at[b], semI[b])

    def issue_stage2(b):
        pltpu.make_async_copy(srcp.at[pl.ds(0, _CA)], sidx.at[b], semI[b]).wait()
        pltpu.make_async_copy(dstp.at[pl.ds(0, _CA)], didx.at[b], semI[b]).wait()
        pltpu.async_copy(ltabS.at[sidx.at[b]], bufS.at[b], semS[b])
        pltpu.async_copy(ltabD.at[didx.at[b]], bufD.at[b], semD[b])

    def finish(j, b):
        base = wid * per_worker + j * _CA
        pltpu.make_async_copy(ltabS.at[sidx.at[b]], bufS.at[b], semS[b]).wait()
        pltpu.make_async_copy(ltabD.at[didx.at[b]], bufD.at[b], semD[b]).wait()

        def edge_body(e):
            z = bufS[b, e] + bufD[b, e]
            z = jnp.where(z > 0, z, 0.2 * z)
            w16 = jnp.where(lane < 8, jnp.exp(z), 0.0)
            wbuf[e] = w16
        pl.loop(0, _CA)(edge_body)

        pltpu.sync_copy(wbuf, w_out.at[pl.ds(base, _CA)])
        pltpu.sync_copy(wbuf, den_sh.at[didx.at[b]], add=True)

    issue_stage1(0, 0)
    issue_stage2(0)
    issue_stage1(1, 1)
    issue_stage2(1)

    def pair_body(m):
        j0 = 2 * m
        finish(j0, 0)

        @pl.when(m + 1 < npairs)
        def _():
            issue_stage1(j0 + 2, 0)
            issue_stage2(0)

        finish(j0 + 1, 1)

        @pl.when(m + 1 < npairs)
        def _():
            issue_stage1(j0 + 3, 1)
            issue_stage2(1)
    pl.loop(0, npairs)(pair_body)

    plsc.subcore_barrier()
    pltpu.sync_copy(den_sh.at[pl.ds(s * rows_per_tile, rows_per_tile)], stage)
    pltpu.sync_copy(stage, den_out.at[c, pl.ds(s * rows_per_tile, rows_per_tile)])


@functools.lru_cache(maxsize=4)
def _make_sc_aggregate(layer, rmult):
    cb = _CB1 if layer == 1 else _CB2
    nrows = 1 if layer == 1 else 8

    @functools.partial(
        pl.kernel,
        out_type=_f32((_NPD, _NC, 128)),
        mesh=_get_mesh(),
        compiler_params=_sc_params,
        scratch_types=dict(
            sidx=pltpu.VMEM((2, cb), jnp.int32),
            didx=pltpu.VMEM((2, cb), jnp.int32),
            gidx=pltpu.VMEM((2, cb * nrows), jnp.int32),
            fbuf=pltpu.VMEM((2, cb * nrows, 128), jnp.float32),
            wbuf=pltpu.VMEM((2, cb, 16), jnp.float32),
            rdbuf=pltpu.VMEM((2, cb, 16), jnp.float32),
            abuf=pltpu.VMEM((16,), jnp.float32),
            obuf=pltpu.VMEM((cb, 128), jnp.float32),
            stage=pltpu.VMEM((32, 128), jnp.float32),
            acc_sh=pltpu.VMEM_SHARED((_NPD, 128), jnp.float32),
            semI0=pltpu.SemaphoreType.DMA,
            semI1=pltpu.SemaphoreType.DMA,
            semR0=pltpu.SemaphoreType.DMA,
            semR1=pltpu.SemaphoreType.DMA,
            semF0=pltpu.SemaphoreType.DMA,
            semF1=pltpu.SemaphoreType.DMA,
        ),
    )
    def _sc_agg(ft, srcp, dstp, w_in, rden, out,
                sidx, didx, gidx, fbuf, wbuf, rdbuf, abuf, obuf, stage,
                acc_sh, semI0, semI1, semR0, semR1, semF0, semF1):
        c = lax.axis_index("c")
        s = lax.axis_index("s")
        per_tile = _EP // _NS
        nchunk = per_tile // cb
        npairs = nchunk // 2
        rows_per_tile = _NPD // _NS
        semI = (semI0, semI1)
        semR = (semR0, semR1)
        semF = (semF0, semF1)

        _zero_vmem(stage, 32, 128)
        def zrow(i):
            pltpu.sync_copy(
                stage, acc_sh.at[pl.ds(s * rows_per_tile + i * 32, 32)])
        pl.loop(0, rows_per_tile // 32)(zrow)
        plsc.subcore_barrier()

        lane = lax.iota(jnp.int32, 16)

        def issue_stage1(j, b):
            base = s * per_tile + j * cb
            pltpu.async_copy(srcp.at[pl.ds(base, cb)], sidx.at[b], semI[b])
            pltpu.async_copy(dstp.at[pl.ds(base, cb)], didx.at[b], semI[b])
            pltpu.async_copy(w_in.at[pl.ds(base, cb)], wbuf.at[b], semI[b])

        def issue_stage2(b):
            pltpu.make_async_copy(srcp.at[pl.ds(0, cb)], sidx.at[b], semI[b]).wait()
            pltpu.make_async_copy(dstp.at[pl.ds(0, cb)], didx.at[b], semI[b]).wait()
            pltpu.make_async_copy(w_in.at[pl.ds(0, cb)], wbuf.at[b], semI[b]).wait()

            def gi_body(i):
                s16 = sidx[b, pl.ds(i * 16, 16)]
                if layer == 1:
                    gidx[b, pl.ds(i * 16, 16)] = s16 * rmult + c
                else:
                    pos0 = (i * 16 + lane) * 8
                    for k in range(8):
                        plsc.store_scatter(gidx.at[b], [pos0 + k],
                                           s16 * rmult + (2 * k + c))
            pl.loop(0, cb // 16)(gi_body)
            pltpu.async_copy(rden.at[didx.at[b]], rdbuf.at[b], semR[b])
            pltpu.async_copy(ft.at[gidx.at[b]], fbuf.at[b], semF[b])

        def finish(b):
            pltpu.make_async_copy(rden.at[didx.at[b]], rdbuf.at[b], semR[b]).wait()
            pltpu.make_async_copy(ft.at[gidx.at[b]], fbuf.at[b], semF[b]).wait()

            def edge_body(e):
                arow = wbuf[b, e] * rdbuf[b, e]
                abuf[...] = arow
                if layer == 1:
                    for v in range(8):
                        h = (c * 8 + v) // 2
                        sv = plsc.load_gather(
                            abuf, [jnp.broadcast_to(h, (16,)).astype(jnp.int32)])
                        obuf[e, pl.ds(v * 16, 16)] = fbuf[b, e, pl.ds(v * 16, 16)] * sv
                else:
                    acc = [jnp.zeros((16,), jnp.float32) for _ in range(8)]
                    for k in range(8):
                        sv = plsc.load_gather(
                            abuf, [jnp.full((16,), k, jnp.int32)])
                        for v in range(8):
                            acc[v] = acc[v] + sv * fbuf[b, e * 8 + k, pl.ds(v * 16, 16)]
                    for v in range(8):
                        obuf[e, pl.ds(v * 16, 16)] = acc[v]
            pl.loop(0, cb)(edge_body)

            pltpu.sync_copy(obuf, acc_sh.at[didx.at[b]], add=True)

        issue_stage1(0, 0)
        issue_stage2(0)
        issue_stage1(1, 1)
        issue_stage2(1)

        def pair_body(m):
            j0 = 2 * m
            finish(0)

            @pl.when(m + 1 < npairs)
            def _():
                issue_stage1(j0 + 2, 0)
                issue_stage2(0)

            finish(1)

            @pl.when(m + 1 < npairs)
            def _():
                issue_stage1(j0 + 3, 1)
                issue_stage2(1)
        pl.loop(0, npairs)(pair_body)

        plsc.subcore_barrier()

        def out_body(i):
            r0 = s * rows_per_tile + i * 32
            pltpu.sync_copy(acc_sh.at[pl.ds(r0, 32)], stage)
            pltpu.sync_copy(stage, out.at[pl.ds(r0, 32), c])
        pl.loop(0, rows_per_tile // 32)(out_body)

    return _sc_agg


def _sc_edge_w(*args):
    return _make_sc_edge_w()(*args)


def _sc_agg1(*args):
    return _make_sc_aggregate(1, 3)(*args)


_CF = 128


@functools.lru_cache(maxsize=1)
def _make_sc_fbuild():
    @functools.partial(
        pl.kernel,
        out_type=(_f32((_EP, _HID)), _f32((_EP, 16))),
        mesh=_get_mesh(),
        compiler_params=_sc_params,
        scratch_types=dict(
            sidx=pltpu.VMEM((2, _CF), jnp.int32),
            didx=pltpu.VMEM((2, _CF), jnp.int32),
            gidx=pltpu.VMEM((2, _CF), jnp.int32),
            fbuf=pltpu.VMEM((2, _CF, 128), jnp.float32),
            wbuf=pltpu.VMEM((2, _CF, 16), jnp.float32),
            rdbuf=pltpu.VMEM((2, _CF, 16), jnp.float32),
            abuf=pltpu.VMEM((_CF, 16), jnp.float32),
            semI0=pltpu.SemaphoreType.DMA,
            semI1=pltpu.SemaphoreType.DMA,
            semR0=pltpu.SemaphoreType.DMA,
            semR1=pltpu.SemaphoreType.DMA,
            semF0=pltpu.SemaphoreType.DMA,
            semF1=pltpu.SemaphoreType.DMA,
            semO0=pltpu.SemaphoreType.DMA,
            semO1=pltpu.SemaphoreType.DMA,
        ),
    )
    def _sc_fbuild(ft, srcp, dstp, w_in, rden, f_out, al_out,
                   sidx, didx, gidx, fbuf, wbuf, rdbuf, abuf,
                   semI0, semI1, semR0, semR1, semF0, semF1, semO0, semO1):
        c = lax.axis_index("c")
        s = lax.axis_index("s")
        per_tile = _EP // _NS
        nchunk = per_tile // _CF
        npairs = nchunk // 2
        semI = (semI0, semI1)
        semR = (semR0, semR1)
        semF = (semF0, semF1)
        semO = (semO0, semO1)

        def issue_stage1(j, b):
            base = s * per_tile + j * _CF
            pltpu.async_copy(srcp.at[pl.ds(base, _CF)], sidx.at[b], semI[b])
            pltpu.async_copy(dstp.at[pl.ds(base, _CF)], didx.at[b], semI[b])
            pltpu.async_copy(w_in.at[pl.ds(base, _CF)], wbuf.at[b], semI[b])

        def issue_stage2(b, drain_write):
            pltpu.make_async_copy(srcp.at[pl.ds(0, _CF)], sidx.at[b], semI[b]).wait()
            pltpu.make_async_copy(dstp.at[pl.ds(0, _CF)], didx.at[b], semI[b]).wait()
            pltpu.make_async_copy(w_in.at[pl.ds(0, _CF)], wbuf.at[b], semI[b]).wait()

            def gi_body(i):
                s16 = sidx[b, pl.ds(i * 16, 16)]
                gidx[b, pl.ds(i * 16, 16)] = s16 * 2 + c
            pl.loop(0, _CF // 16)(gi_body)
            pltpu.async_copy(rden.at[didx.at[b]], rdbuf.at[b], semR[b])
            if drain_write:
                pltpu.make_async_copy(
                    fbuf.at[b],
                    f_out.at[pl.ds(0, _CF), pl.ds(c * 128, 128)],
                    semO[b]).wait()
            pltpu.async_copy(ft.at[gidx.at[b]], fbuf.at[b], semF[b])

        def finish(j, b):
            base = s * per_tile + j * _CF
            pltpu.make_async_copy(rden.at[didx.at[b]], rdbuf.at[b], semR[b]).wait()
            pltpu.make_async_copy(ft.at[gidx.at[b]], fbuf.at[b], semF[b]).wait()

            def edge_body(e):
                abuf[e] = wbuf[b, e] * rdbuf[b, e]
            pl.loop(0, _CF)(edge_body)

            pltpu.async_copy(fbuf.at[b],
                             f_out.at[pl.ds(base, _CF), pl.ds(c * 128, 128)],
                             semO[b])

            @pl.when(c == 0)
            def _():
                pltpu.sync_copy(abuf, al_out.at[pl.ds(base, _CF)])

        issue_stage1(0, 0)
        issue_stage2(0, False)
        issue_stage1(1, 1)
        issue_stage2(1, False)

        def pair_body(m):
            j0 = 2 * m
            finish(j0, 0)

            @pl.when(m + 1 < npairs)
            def _():
                issue_stage1(j0 + 2, 0)
                issue_stage2(0, True)

            finish(j0 + 1, 1)

            @pl.when(m + 1 < npairs)
            def _():
                issue_stage1(j0 + 3, 1)
                issue_stage2(1, True)
        pl.loop(0, npairs)(pair_body)

        pltpu.make_async_copy(
            fbuf.at[0], f_out.at[pl.ds(0, _CF), pl.ds(c * 128, 128)],
            semO[0]).wait()
        pltpu.make_async_copy(
            fbuf.at[1], f_out.at[pl.ds(0, _CF), pl.ds(c * 128, 128)],
            semO[1]).wait()

    return _sc_fbuild


@functools.lru_cache(maxsize=1)
def _make_sc_segsum():
    @functools.partial(
        pl.kernel,
        out_type=_f32((_NPD, _NC, 128)),
        mesh=_get_mesh(),
        compiler_params=_sc_params,
        scratch_types=dict(
            didx=pltpu.VMEM((2, _CF), jnp.int32),
            gbuf=pltpu.VMEM((2, _CF, 128), jnp.float32),
            stage=pltpu.VMEM((32, 128), jnp.float32),
            acc_sh=pltpu.VMEM_SHARED((_NPD, 128), jnp.float32),
            semI0=pltpu.SemaphoreType.DMA,
            semI1=pltpu.SemaphoreType.DMA,
            semG0=pltpu.SemaphoreType.DMA,
            semG1=pltpu.SemaphoreType.DMA,
        ),
    )
    def _sc_segsum(g2, dstp, out, didx, gbuf, stage, acc_sh,
                   semI0, semI1, semG0, semG1):
        c = lax.axis_index("c")
        s = lax.axis_index("s")
        per_tile = _EP // _NS
        nchunk = per_tile // _CF
        npairs = nchunk // 2
        rows_per_tile = _NPD // _NS
        semI = (semI0, semI1)
        semG = (semG0, semG1)

        _zero_vmem(stage, 32, 128)
        def zrow(i):
            pltpu.sync_copy(
                stage, acc_sh.at[pl.ds(s * rows_per_tile + i * 32, 32)])
        pl.loop(0, rows_per_tile // 32)(zrow)
        plsc.subcore_barrier()

        def issue(j, b):
            base = s * per_tile + j * _CF
            pltpu.async_copy(dstp.at[pl.ds(base, _CF)], didx.at[b], semI[b])
            pltpu.async_copy(g2.at[pl.ds(base, _CF), pl.ds(c * 128, 128)],
                             gbuf.at[b], semG[b])

        def finish(b):
            pltpu.make_async_copy(dstp.at[pl.ds(0, _CF)], didx.at[b], semI[b]).wait()
            pltpu.make_async_copy(g2.at[pl.ds(0, _CF), pl.ds(c * 128, 128)],
                                  gbuf.at[b], semG[b]).wait()
            pltpu.sync_copy(gbuf.at[b], acc_sh.at[didx.at[b]], add=True)

        issue(0, 0)
        issue(1, 1)

        def pair_body(m):
            j0 = 2 * m
            finish(0)

            @pl.when(m + 1 < npairs)
            def _():
                issue(j0 + 2, 0)

            finish(1)

            @pl.when(m + 1 < npairs)
            def _():
                issue(j0 + 3, 1)
        pl.loop(0, npairs)(pair_body)

        plsc.subcore_barrier()

        def out_body(i):
            r0 = s * rows_per_tile + i * 32
            pltpu.sync_copy(acc_sh.at[pl.ds(r0, 32)], stage)
            pltpu.sync_copy(stage, out.at[pl.ds(r0, 32), c])
        pl.loop(0, rows_per_tile // 32)(out_body)

    return _sc_segsum


def _gcomb_body(f_ref, al_ref, w2_ref, b1w2_ref, o_ref):
    f = f_ref[...].astype(jnp.bfloat16)
    t = (jnp.dot(f, w2_ref[...], preferred_element_type=jnp.float32)
         + b1w2_ref[...])
    al = al_ref[...]
    g = al[:, 0:1] * t[:, 0:_HID]
    for k in range(1, _HEADS):
        g = g + al[:, k:k + 1] * t[:, k * _HID:(k + 1) * _HID]
    o_ref[...] = g


def _gcomb(F, alpha, W2bf, b1w2, blk=512):
    return pl.pallas_call(
        _gcomb_body,
        grid=(_EP // blk,),
        in_specs=[
            pl.BlockSpec((blk, _HID), lambda i: (i, 0)),
            pl.BlockSpec((blk, 16), lambda i: (i, 0)),
            pl.BlockSpec((_HID, _HEADS * _HID), lambda i: (0, 0)),
            pl.BlockSpec((1, _HEADS * _HID), lambda i: (0, 0)),
        ],
        out_specs=pl.BlockSpec((blk, _HID), lambda i: (i, 0)),
        out_shape=_f32((_EP, _HID)),
    )(F, alpha, W2bf, b1w2.reshape(1, -1))


def _pad_rows(a, rows):
    return jnp.pad(a, ((0, rows - a.shape[0]), (0, 0)))


def _gat_edge_pass(lg, ft_flat, srcp, dstp, agg_fn):
    ltabS = jnp.concatenate([lg[:, 0:8], lg[:, 0:8]], axis=1)
    ltabD = jnp.concatenate([lg[:, 8:16], lg[:, 8:16]], axis=1)
    w, denp = _sc_edge_w(ltabS, ltabD, srcp, dstp)
    rden = _rden(denp)
    out = agg_fn(ft_flat, srcp, dstp, w, rden)
    return out.reshape(_NPD, 256)


def kernel(x, W1, a_src1, a_dst1, b1, W2, a_src2, a_dst2, b2,
           Wt1, bt1, Wt2, bt2, Wc1, bc1, Wc2, bc2,
           edge_index, timestamp_target, node_target, node_mask):
    n = _N
    npad = _EP - _E - n
    loop = jnp.arange(n, dtype=edge_index.dtype)
    pad_src = jnp.full((npad,), n, dtype=edge_index.dtype)
    pad_dst = n + (jnp.arange(npad, dtype=edge_index.dtype) % (_NPD - n))
    srcp = jnp.concatenate([edge_index[0], loop, pad_src])
    dstp = jnp.concatenate([edge_index[1], loop, pad_dst])

    d1 = _HID // _HEADS
    c1s = jnp.einsum("dkc,kc->dk", W1.reshape(_D_IN, _HEADS, d1), a_src1)
    c1d = jnp.einsum("dkc,kc->dk", W1.reshape(_D_IN, _HEADS, d1), a_dst1)
    c2s = jnp.einsum("dkc,kc->dk", W2.reshape(_HID, _HEADS, _HID), a_src2)
    c2d = jnp.einsum("dkc,kc->dk", W2.reshape(_HID, _HEADS, _HID), a_dst2)
    W1cat = jnp.concatenate(
        [W1, c1s, c1d, jnp.zeros((_D_IN, 112), jnp.float32)], axis=1)
    bias1 = jnp.zeros((384,), jnp.float32)
    W2log = jnp.concatenate(
        [c2s, c2d, jnp.zeros((_HID, 112), jnp.float32)], axis=1)
    b2log = b1 @ W2log
    W2bf = W2.astype(jnp.bfloat16)
    b1w2 = b1 @ W2

    xp = _pad_rows(x, _NPD)

    hl = _mm(xp, W1cat, bias1)
    lg1 = hl[:, 256:272]
    out1 = _gat_edge_pass(lg1, hl.reshape(_NPD * 3, 128), srcp, dstp, _sc_agg1)

    lgp = _mm(out1, W2log, b2log)
    lg2 = lgp[:, 0:16]
    ltabS2 = jnp.concatenate([lg2[:, 0:8], lg2[:, 0:8]], axis=1)
    ltabD2 = jnp.concatenate([lg2[:, 8:16], lg2[:, 8:16]], axis=1)
    w2e, denp2 = _sc_edge_w(ltabS2, ltabD2, srcp, dstp)
    rden2 = _rden(denp2)
    F, alpha = _make_sc_fbuild()(out1.reshape(_NPD * 2, 128), srcp, dstp,
                                 w2e, rden2)
    G = _gcomb(F, alpha, W2bf, b1w2)
    out2 = _make_sc_segsum()(G, dstp).reshape(_NPD, 256)

    wt2p = jnp.zeros((_HID, 128), jnp.float32).at[:, 0:_T_OUT].set(Wt2)
    bt2p = jnp.zeros((128,), jnp.float32).at[0:_T_OUT].set(bt2)
    wc2p = jnp.zeros((_HID, 128), jnp.float32).at[:, 0:2].set(Wc2)
    bc2p = jnp.zeros((128,), jnp.float32).at[0:2].set(bc2)
    meta = jnp.zeros((_NPD, 128), jnp.float32)
    meta = meta.at[:n, 0].set(timestamp_target.astype(jnp.float32))
    meta = meta.at[:n, 1].set(node_target.astype(jnp.float32))
    meta = meta.at[:n, 2].set(node_mask.astype(jnp.float32))
    meta = meta.at[:n, 3].set(1.0)

    sums = _head_loss(out2, b2, Wt1, bt1, wt2p, bt2p, Wc1, bc1, wc2p, bc2p, meta)
    ts_loss = sums[0, 0] / jnp.float32(n)
    cls_loss = sums[0, 1] / jnp.maximum(sums[0, 2], 1.0)
    return cls_loss + ts_loss


# speedup vs baseline: 1.1023x; 1.1023x over previous
import functools

import jax
import jax.numpy as jnp
from jax import lax
from jax.experimental import pallas as pl
from jax.experimental.pallas import tpu as pltpu
from jax.experimental.pallas import tpu_sc as plsc

_N = 10000
_E = 320000
_D_IN = 128
_HID = 256
_HEADS = 8
_T_OUT = 5
_NPD = 10240
_EP = 344064
_NC = 2
_NS = 16
_CA = 256
_CB1 = 64
_CB2 = 16

_sc_params = pltpu.CompilerParams(use_tc_tiling_on_sc=False,
                                  needs_layout_passes=False)


@functools.lru_cache(maxsize=1)
def _get_mesh():
    return plsc.VectorSubcoreMesh(core_axis_name="c", subcore_axis_name="s",
                                  num_cores=_NC, num_subcores=_NS)


def _f32(shape):
    return jax.ShapeDtypeStruct(shape, jnp.float32)


def _mm_body(a_ref, b_ref, bias_ref, o_ref):
    o_ref[...] = (jnp.dot(a_ref[...], b_ref[...], preferred_element_type=jnp.float32)
                  + bias_ref[...])


def _mm(a, b, bias, blk=1024):
    m, k = a.shape
    _, n = b.shape
    return pl.pallas_call(
        _mm_body,
        grid=(m // blk,),
        in_specs=[
            pl.BlockSpec((blk, k), lambda i: (i, 0)),
            pl.BlockSpec((k, n), lambda i: (0, 0)),
            pl.BlockSpec((1, n), lambda i: (0, 0)),
        ],
        out_specs=pl.BlockSpec((blk, n), lambda i: (i, 0)),
        out_shape=_f32((m, n)),
    )(a, b, bias.reshape(1, n))


def _rden_body(d_ref, o_ref):
    s = d_ref[0] + d_ref[1]
    o_ref[...] = 1.0 / jnp.maximum(s, 1e-30)


def _rden(denp):
    d = denp.reshape(2, _NPD * 16 // 128, 128)
    r = pl.pallas_call(
        _rden_body,
        grid=(4,),
        in_specs=[pl.BlockSpec((2, _NPD * 4 // 128, 128), lambda i: (0, i, 0))],
        out_specs=pl.BlockSpec((_NPD * 4 // 128, 128), lambda i: (i, 0)),
        out_shape=_f32((_NPD * 16 // 128, 128)),
    )(d)
    return r.reshape(_NPD, 16)


def _head_loss_body(h2_ref, b2_ref, wt1_ref, bt1_ref, wt2_ref, bt2_ref,
                    wc1_ref, bc1_ref, wc2_ref, bc2_ref, meta_ref, o_ref):
    i = pl.program_id(0)
    h2 = h2_ref[...] * (1.0 / _HEADS) + b2_ref[...]
    t1 = jnp.maximum(jnp.dot(h2, wt1_ref[...], preferred_element_type=jnp.float32)
                     + bt1_ref[...], 0.0)
    tl = jnp.dot(t1, wt2_ref[...], preferred_element_type=jnp.float32) + bt2_ref[...]
    c1 = jnp.maximum(jnp.dot(h2, wc1_ref[...], preferred_element_type=jnp.float32)
                     + bc1_ref[...], 0.0)
    cl = jnp.dot(c1, wc2_ref[...], preferred_element_type=jnp.float32) + bc2_ref[...]
    meta = meta_ref[...]
    tt = meta[:, 0:1]
    nt = meta[:, 1:2]
    msk = meta[:, 2:3]
    valid = meta[:, 3:4]
    colid = lax.broadcasted_iota(jnp.int32, tl.shape, 1).astype(jnp.float32)
    neg = jnp.float32(-1e30)

    tl5 = jnp.where(colid < 5.0, tl, neg)
    tmax = jnp.max(tl5, axis=1, keepdims=True)
    tsum = jnp.sum(jnp.where(colid < 5.0, jnp.exp(tl5 - tmax), 0.0), axis=1, keepdims=True)
    tlse = tmax + jnp.log(tsum)
    tsel = jnp.sum(jnp.where(colid == tt, tl, 0.0), axis=1, keepdims=True)
    ts_ce = (tlse - tsel) * valid

    cl2 = jnp.where(colid < 2.0, cl, neg)
    cmax = jnp.max(cl2, axis=1, keepdims=True)
    csum = jnp.sum(jnp.where(colid < 2.0, jnp.exp(cl2 - cmax), 0.0), axis=1, keepdims=True)
    clse = cmax + jnp.log(csum)
    csel = jnp.sum(jnp.where(colid == nt, cl, 0.0), axis=1, keepdims=True)
    cls_ce = (clse - csel) * msk * valid

    ts_s = jnp.sum(ts_ce)
    cls_s = jnp.sum(cls_ce)
    m_s = jnp.sum(msk * valid)
    ci = lax.broadcasted_iota(jnp.int32, (1, 128), 1).astype(jnp.float32)
    row = (jnp.where(ci == 0.0, ts_s, 0.0)
           + jnp.where(ci == 1.0, cls_s, 0.0)
           + jnp.where(ci == 2.0, m_s, 0.0))

    @pl.when(i == 0)
    def _():
        o_ref[...] = row

    @pl.when(i > 0)
    def _():
        o_ref[...] = o_ref[...] + row


def _head_loss(h2p, b2, wt1, bt1, wt2p, bt2p, wc1, bc1, wc2p, bc2p, meta, blk=1024):
    vec = lambda v: v.reshape(1, -1)
    return pl.pallas_call(
        _head_loss_body,
        grid=(_NPD // blk,),
        in_specs=[
            pl.BlockSpec((blk, _HID), lambda i: (i, 0)),
            pl.BlockSpec((1, _HID), lambda i: (0, 0)),
            pl.BlockSpec((_HID, _HID), lambda i: (0, 0)),
            pl.BlockSpec((1, _HID), lambda i: (0, 0)),
            pl.BlockSpec((_HID, 128), lambda i: (0, 0)),
            pl.BlockSpec((1, 128), lambda i: (0, 0)),
            pl.BlockSpec((_HID, _HID), lambda i: (0, 0)),
            pl.BlockSpec((1, _HID), lambda i: (0, 0)),
            pl.BlockSpec((_HID, 128), lambda i: (0, 0)),
            pl.BlockSpec((1, 128), lambda i: (0, 0)),
            pl.BlockSpec((blk, 128), lambda i: (i, 0)),
        ],
        out_specs=pl.BlockSpec((1, 128), lambda i: (0, 0)),
        out_shape=_f32((1, 128)),
    )(h2p, vec(b2), wt1, vec(bt1), wt2p, vec(bt2p), wc1, vec(bc1), wc2p, vec(bc2p), meta)


def _zero_vmem(ref, rows, width):
    z = jnp.zeros((16,), jnp.float32)
    def body(i):
        r = i // (width // 16)
        v = i % (width // 16)
        ref[r, pl.ds(v * 16, 16)] = z
    pl.loop(0, rows * (width // 16))(body)


@functools.lru_cache(maxsize=1)
def _make_sc_edge_w():
    return functools.partial(
        pl.kernel,
        out_type=(_f32((_EP, 16)), _f32((_NC, _NPD, 16))),
        mesh=_get_mesh(),
        compiler_params=_sc_params,
        scratch_types=dict(
            sidx=pltpu.VMEM((2, _CA), jnp.int32),
            didx=pltpu.VMEM((2, _CA), jnp.int32),
            bufS=pltpu.VMEM((2, _CA, 16), jnp.float32),
            bufD=pltpu.VMEM((2, _CA, 16), jnp.float32),
            wbuf=pltpu.VMEM((_CA, 16), jnp.float32),
            stage=pltpu.VMEM((_NPD // _NS, 16), jnp.float32),
            den_sh=pltpu.VMEM_SHARED((_NPD, 16), jnp.float32),
            semI0=pltpu.SemaphoreType.DMA,
            semI1=pltpu.SemaphoreType.DMA,
            semS0=pltpu.SemaphoreType.DMA,
            semS1=pltpu.SemaphoreType.DMA,
            semD0=pltpu.SemaphoreType.DMA,
            semD1=pltpu.SemaphoreType.DMA,
        ),
    )(_sc_edge_w_body)


def _sc_edge_w_body(ltabS, ltabD, srcp, dstp, w_out, den_out,
                    sidx, didx, bufS, bufD, wbuf, stage, den_sh,
                    semI0, semI1, semS0, semS1, semD0, semD1):
    c = lax.axis_index("c")
    s = lax.axis_index("s")
    wid = s * _NC + c
    per_worker = _EP // (_NC * _NS)
    nchunk = per_worker // _CA
    npairs = nchunk // 2
    rows_per_tile = _NPD // _NS
    semI = (semI0, semI1)
    semS = (semS0, semS1)
    semD = (semD0, semD1)

    _zero_vmem(stage, rows_per_tile, 16)
    pltpu.sync_copy(stage, den_sh.at[pl.ds(s * rows_per_tile, rows_per_tile)])
    plsc.subcore_barrier()

    lane = lax.iota(jnp.int32, 16)

    def issue_stage1(j, b):
        base = wid * per_worker + j * _CA
        pltpu.async_copy(srcp.at[pl.ds(base, _CA)], sidx.at[b], semI[b])
        pltpu.async_copy(dstp.at[pl.ds(base, _CA)], didx.at[b], semI[b])

    def issue_stage2(b):
        pltpu.make_async_copy(srcp.at[pl.ds(0, _CA)], sidx.at[b], semI[b]).wait()
        pltpu.make_async_copy(dstp.at[pl.ds(0, _CA)], didx.at[b], semI[b]).wait()
        pltpu.async_copy(ltabS.at[sidx.at[b]], bufS.at[b], semS[b])
        pltpu.async_copy(ltabD.at[didx.at[b]], bufD.at[b], semD[b])

    def finish(j, b):
        base = wid * per_worker + j * _CA
        pltpu.make_async_copy(ltabS.at[sidx.at[b]], bufS.at[b], semS[b]).wait()
        pltpu.make_async_copy(ltabD.at[didx.at[b]], bufD.at[b], semD[b]).wait()

        def edge_body(e):
            z = bufS[b, e] + bufD[b, e]
            z = jnp.where(z > 0, z, 0.2 * z)
            w16 = jnp.where(lane < 8, jnp.exp(z), 0.0)
            wbuf[e] = w16
        pl.loop(0, _CA)(edge_body)

        pltpu.sync_copy(wbuf, w_out.at[pl.ds(base, _CA)])
        pltpu.sync_copy(wbuf, den_sh.at[didx.at[b]], add=True)

    issue_stage1(0, 0)
    issue_stage2(0)
    issue_stage1(1, 1)
    issue_stage2(1)

    def pair_body(m):
        j0 = 2 * m
        finish(j0, 0)

        @pl.when(m + 1 < npairs)
        def _():
            issue_stage1(j0 + 2, 0)
            issue_stage2(0)

        finish(j0 + 1, 1)

        @pl.when(m + 1 < npairs)
        def _():
            issue_stage1(j0 + 3, 1)
            issue_stage2(1)
    pl.loop(0, npairs)(pair_body)

    plsc.subcore_barrier()
    pltpu.sync_copy(den_sh.at[pl.ds(s * rows_per_tile, rows_per_tile)], stage)
    pltpu.sync_copy(stage, den_out.at[c, pl.ds(s * rows_per_tile, rows_per_tile)])


@functools.lru_cache(maxsize=4)
def _make_sc_aggregate(layer, rmult):
    cb = _CB1 if layer == 1 else _CB2
    nrows = 1 if layer == 1 else 8

    @functools.partial(
        pl.kernel,
        out_type=_f32((_NPD, _NC, 128)),
        mesh=_get_mesh(),
        compiler_params=_sc_params,
        scratch_types=dict(
            sidx=pltpu.VMEM((2, cb), jnp.int32),
            didx=pltpu.VMEM((2, cb), jnp.int32),
            gidx=pltpu.VMEM((2, cb * nrows), jnp.int32),
            fbuf=pltpu.VMEM((2, cb * nrows, 128), jnp.float32),
            wbuf=pltpu.VMEM((2, cb, 16), jnp.float32),
            rdbuf=pltpu.VMEM((2, cb, 16), jnp.float32),
            abuf=pltpu.VMEM((16,), jnp.float32),
            obuf=pltpu.VMEM((cb, 128), jnp.float32),
            stage=pltpu.VMEM((32, 128), jnp.float32),
            acc_sh=pltpu.VMEM_SHARED((_NPD, 128), jnp.float32),
            semI0=pltpu.SemaphoreType.DMA,
            semI1=pltpu.SemaphoreType.DMA,
            semR0=pltpu.SemaphoreType.DMA,
            semR1=pltpu.SemaphoreType.DMA,
            semF0=pltpu.SemaphoreType.DMA,
            semF1=pltpu.SemaphoreType.DMA,
        ),
    )
    def _sc_agg(ft, srcp, dstp, w_in, rden, out,
                sidx, didx, gidx, fbuf, wbuf, rdbuf, abuf, obuf, stage,
                acc_sh, semI0, semI1, semR0, semR1, semF0, semF1):
        c = lax.axis_index("c")
        s = lax.axis_index("s")
        per_tile = _EP // _NS
        nchunk = per_tile // cb
        npairs = nchunk // 2
        rows_per_tile = _NPD // _NS
        semI = (semI0, semI1)
        semR = (semR0, semR1)
        semF = (semF0, semF1)

        _zero_vmem(stage, 32, 128)
        def zrow(i):
            pltpu.sync_copy(
                stage, acc_sh.at[pl.ds(s * rows_per_tile + i * 32, 32)])
        pl.loop(0, rows_per_tile // 32)(zrow)
        plsc.subcore_barrier()

        lane = lax.iota(jnp.int32, 16)

        def issue_stage1(j, b):
            base = s * per_tile + j * cb
            pltpu.async_copy(srcp.at[pl.ds(base, cb)], sidx.at[b], semI[b])
            pltpu.async_copy(dstp.at[pl.ds(base, cb)], didx.at[b], semI[b])
            pltpu.async_copy(w_in.at[pl.ds(base, cb)], wbuf.at[b], semI[b])

        def issue_stage2(b):
            pltpu.make_async_copy(srcp.at[pl.ds(0, cb)], sidx.at[b], semI[b]).wait()
            pltpu.make_async_copy(dstp.at[pl.ds(0, cb)], didx.at[b], semI[b]).wait()
            pltpu.make_async_copy(w_in.at[pl.ds(0, cb)], wbuf.at[b], semI[b]).wait()

            def gi_body(i):
                s16 = sidx[b, pl.ds(i * 16, 16)]
                if layer == 1:
                    gidx[b, pl.ds(i * 16, 16)] = s16 * rmult + c
                else:
                    pos0 = (i * 16 + lane) * 8
                    for k in range(8):
                        plsc.store_scatter(gidx.at[b], [pos0 + k],
                                           s16 * rmult + (2 * k + c))
            pl.loop(0, cb // 16)(gi_body)
            pltpu.async_copy(rden.at[didx.at[b]], rdbuf.at[b], semR[b])
            pltpu.async_copy(ft.at[gidx.at[b]], fbuf.at[b], semF[b])

        def finish(b):
            pltpu.make_async_copy(rden.at[didx.at[b]], rdbuf.at[b], semR[b]).wait()
            pltpu.make_async_copy(ft.at[gidx.at[b]], fbuf.at[b], semF[b]).wait()

            def edge_body(e):
                arow = wbuf[b, e] * rdbuf[b, e]
                abuf[...] = arow
                if layer == 1:
                    for v in range(8):
                        h = (c * 8 + v) // 2
                        sv = plsc.load_gather(
                            abuf, [jnp.broadcast_to(h, (16,)).astype(jnp.int32)])
                        obuf[e, pl.ds(v * 16, 16)] = fbuf[b, e, pl.ds(v * 16, 16)] * sv
                else:
                    acc = [jnp.zeros((16,), jnp.float32) for _ in range(8)]
                    for k in range(8):
                        sv = plsc.load_gather(
                            abuf, [jnp.full((16,), k, jnp.int32)])
                        for v in range(8):
                            acc[v] = acc[v] + sv * fbuf[b, e * 8 + k, pl.ds(v * 16, 16)]
                    for v in range(8):
                        obuf[e, pl.ds(v * 16, 16)] = acc[v]
            pl.loop(0, cb)(edge_body)

            pltpu.sync_copy(obuf, acc_sh.at[didx.at[b]], add=True)

        issue_stage1(0, 0)
        issue_stage2(0)
        issue_stage1(1, 1)
        issue_stage2(1)

        def pair_body(m):
            j0 = 2 * m
            finish(0)

            @pl.when(m + 1 < npairs)
            def _():
                issue_stage1(j0 + 2, 0)
                issue_stage2(0)

            finish(1)

            @pl.when(m + 1 < npairs)
            def _():
                issue_stage1(j0 + 3, 1)
                issue_stage2(1)
        pl.loop(0, npairs)(pair_body)

        plsc.subcore_barrier()

        def out_body(i):
            r0 = s * rows_per_tile + i * 32
            pltpu.sync_copy(acc_sh.at[pl.ds(r0, 32)], stage)
            pltpu.sync_copy(stage, out.at[pl.ds(r0, 32), c])
        pl.loop(0, rows_per_tile // 32)(out_body)

    return _sc_agg


def _sc_edge_w(*args):
    return _make_sc_edge_w()(*args)


def _sc_agg1(*args):
    return _make_sc_aggregate(1, 3)(*args)


def _sc_agg2(*args):
    return _make_sc_aggregate(2, 17)(*args)


def _pad_rows(a, rows):
    return jnp.pad(a, ((0, rows - a.shape[0]), (0, 0)))


def _gat_edge_pass(lg, ft_flat, srcp, dstp, agg_fn):
    ltabS = jnp.concatenate([lg[:, 0:8], lg[:, 0:8]], axis=1)
    ltabD = jnp.concatenate([lg[:, 8:16], lg[:, 8:16]], axis=1)
    w, denp = _sc_edge_w(ltabS, ltabD, srcp, dstp)
    rden = _rden(denp)
    out = agg_fn(ft_flat, srcp, dstp, w, rden)
    return out.reshape(_NPD, 256)


def kernel(x, W1, a_src1, a_dst1, b1, W2, a_src2, a_dst2, b2,
           Wt1, bt1, Wt2, bt2, Wc1, bc1, Wc2, bc2,
           edge_index, timestamp_target, node_target, node_mask):
    n = _N
    npad = _EP - _E - n
    loop = jnp.arange(n, dtype=edge_index.dtype)
    pad_src = jnp.full((npad,), n, dtype=edge_index.dtype)
    pad_dst = n + (jnp.arange(npad, dtype=edge_index.dtype) % (_NPD - n))
    srcp = jnp.concatenate([edge_index[0], loop, pad_src])
    dstp = jnp.concatenate([edge_index[1], loop, pad_dst])

    d1 = _HID // _HEADS
    c1s = jnp.einsum("dkc,kc->dk", W1.reshape(_D_IN, _HEADS, d1), a_src1)
    c1d = jnp.einsum("dkc,kc->dk", W1.reshape(_D_IN, _HEADS, d1), a_dst1)
    c2s = jnp.einsum("dkc,kc->dk", W2.reshape(_HID, _HEADS, _HID), a_src2)
    c2d = jnp.einsum("dkc,kc->dk", W2.reshape(_HID, _HEADS, _HID), a_dst2)
    W1cat = jnp.concatenate(
        [W1, c1s, c1d, jnp.zeros((_D_IN, 112), jnp.float32)], axis=1)
    W2cat = jnp.concatenate(
        [W2, c2s, c2d, jnp.zeros((_HID, 112), jnp.float32)], axis=1)
    bias1 = jnp.zeros((384,), jnp.float32)
    bias2 = b1 @ W2cat

    xp = _pad_rows(x, _NPD)

    hl = _mm(xp, W1cat, bias1)
    lg1 = hl[:, 256:272]
    out1 = _gat_edge_pass(lg1, hl.reshape(_NPD * 3, 128), srcp, dstp, _sc_agg1)

    pl2 = _mm(out1, W2cat, bias2)
    lg2 = pl2[:, 2048:2064]
    out2 = _gat_edge_pass(lg2, pl2.reshape(_NPD * 17, 128), srcp, dstp, _sc_agg2)

    wt2p = jnp.zeros((_HID, 128), jnp.float32).at[:, 0:_T_OUT].set(Wt2)
    bt2p = jnp.zeros((128,), jnp.float32).at[0:_T_OUT].set(bt2)
    wc2p = jnp.zeros((_HID, 128), jnp.float32).at[:, 0:2].set(Wc2)
    bc2p = jnp.zeros((128,), jnp.float32).at[0:2].set(bc2)
    meta = jnp.zeros((_NPD, 128), jnp.float32)
    meta = meta.at[:n, 0].set(timestamp_target.astype(jnp.float32))
    meta = meta.at[:n, 1].set(node_target.astype(jnp.float32))
    meta = meta.at[:n, 2].set(node_mask.astype(jnp.float32))
    meta = meta.at[:n, 3].set(1.0)

    sums = _head_loss(out2, b2, Wt1, bt1, wt2p, bt2p, Wc1, bc1, wc2p, bc2p, meta)
    ts_loss = sums[0, 0] / jnp.float32(n)
    cls_loss = sums[0, 1] / jnp.maximum(sums[0, 2], 1.0)
    return cls_loss + ts_loss
